# baseline (speedup 1.0000x reference)
import functools
import jax
import jax.numpy as jnp
from jax import lax
from jax.experimental import pallas as pl
from jax.experimental.pallas import tpu as pltpu
from jax.experimental.pallas import tpu_sc as plsc

BATCH = 16384
FEAT = 128
SHAPE_DIM = 300
OUT_DIM = SHAPE_DIM + FEAT
ROW_PAD = 384
TAIL = OUT_DIM - ROW_PAD
SPLIT = ROW_PAD - SHAPE_DIM

NC = 2
NS = 16
NW = NC * NS
B_PER_W = BATCH // NW
CHUNK = 128
NCHUNK = B_PER_W // CHUNK

_mesh = plsc.VectorSubcoreMesh(core_axis_name="c", subcore_axis_name="s")


def _rotate4(v):
    idx = (lax.iota(jnp.int32, 16) + 4) % 16
    return lax.gather(
        v, idx[:, None],
        dimension_numbers=lax.GatherDimensionNumbers(
            offset_dims=(), collapsed_slice_dims=(0,), start_index_map=(0,)),
        slice_sizes=(1,),
        mode=lax.GatherScatterMode.PROMISE_IN_BOUNDS)


@functools.partial(
    pl.kernel,
    mesh=_mesh,
    out_type=jax.ShapeDtypeStruct((BATCH, OUT_DIM), jnp.float32),
    compiler_params=pltpu.CompilerParams(
        skip_device_barrier=True,
        disable_bounds_checks=True,
        disable_semaphore_checks=True,
    ),
    scratch_types=[
        pltpu.VMEM((CHUNK,), jnp.int32),
        pltpu.VMEM((CHUNK, ROW_PAD), jnp.float32),
        pltpu.VMEM((CHUNK, FEAT), jnp.float32),
        pltpu.VMEM((CHUNK, TAIL), jnp.float32),
        pltpu.SemaphoreType.DMA,
    ],
)
def _concat_shape(inputs_hbm, idx_hbm, table_hbm, out_hbm,
                  idx_v, buf_v, inp_v, tail_v, sem):
    wid = lax.axis_index("s") * NC + lax.axis_index("c")
    for c in range(NCHUNK):
        base = wid * B_PER_W + c * CHUNK
        pltpu.sync_copy(idx_hbm.at[pl.ds(base, CHUNK)], idx_v)
        gather = pltpu.async_copy(
            table_hbm.at[idx_v, pl.ds(0, ROW_PAD)],
            buf_v.at[:, pl.ds(0, ROW_PAD)],
            sem,
        )
        pltpu.sync_copy(inputs_hbm.at[pl.ds(base, CHUNK)], inp_v)
        gather.wait()

        lanes = lax.iota(jnp.int32, 16)

        def body(i, carry):
            vg = buf_v[i, pl.ds(288, 16)]
            rot = _rotate4(inp_v[i, pl.ds(0, 16)])
            buf_v[i, pl.ds(288, 16)] = jnp.where(lanes < 12, vg, rot)
            for u in range(5):
                buf_v[i, pl.ds(304 + 16 * u, 16)] = inp_v[i, pl.ds(4 + 16 * u, 16)]
            tail_v[i, pl.ds(0, 16)] = inp_v[i, pl.ds(SPLIT, 16)]
            tail_v[i, pl.ds(28, 16)] = inp_v[i, pl.ds(112, 16)]
            tail_v[i, pl.ds(16, 16)] = inp_v[i, pl.ds(100, 16)]
            return carry
        lax.fori_loop(0, CHUNK, body, 0)

        pltpu.sync_copy(buf_v, out_hbm.at[pl.ds(base, CHUNK), pl.ds(0, ROW_PAD)])
        pltpu.sync_copy(tail_v,
                        out_hbm.at[pl.ds(base, CHUNK), pl.ds(ROW_PAD, TAIL)])


def kernel(inputs, subject_id, s):
    return _concat_shape(inputs, subject_id.astype(jnp.int32), s)


# speedup vs baseline: 1.0546x; 1.0546x over previous
import functools
import jax
import jax.numpy as jnp
from jax import lax
from jax.experimental import pallas as pl
from jax.experimental.pallas import tpu as pltpu
from jax.experimental.pallas import tpu_sc as plsc

BATCH = 16384
FEAT = 128
SHAPE_DIM = 300
OUT_DIM = SHAPE_DIM + FEAT
ROW_PAD = 384
TAIL = OUT_DIM - ROW_PAD
SPLIT = ROW_PAD - SHAPE_DIM

NC = 2
NS = 16
NW = NC * NS
B_PER_W = BATCH // NW
CHUNK = 64
NCHUNK = B_PER_W // CHUNK
NBUF = 2

_mesh = plsc.VectorSubcoreMesh(core_axis_name="c", subcore_axis_name="s")


def _rotate4(v):
    idx = (lax.iota(jnp.int32, 16) + 4) % 16
    return lax.gather(
        v, idx[:, None],
        dimension_numbers=lax.GatherDimensionNumbers(
            offset_dims=(), collapsed_slice_dims=(0,), start_index_map=(0,)),
        slice_sizes=(1,),
        mode=lax.GatherScatterMode.PROMISE_IN_BOUNDS)


@functools.partial(
    pl.kernel,
    mesh=_mesh,
    out_type=jax.ShapeDtypeStruct((BATCH, OUT_DIM), jnp.float32),
    compiler_params=pltpu.CompilerParams(
        skip_device_barrier=True,
        disable_bounds_checks=True,
        disable_semaphore_checks=True,
    ),
    scratch_types=[
        pltpu.VMEM((NBUF, CHUNK), jnp.int32),
        pltpu.VMEM((NBUF, CHUNK, ROW_PAD), jnp.float32),
        pltpu.VMEM((NBUF, CHUNK, FEAT), jnp.float32),
        pltpu.VMEM((NBUF, CHUNK, TAIL), jnp.float32),
        pltpu.SemaphoreType.DMA((NBUF,)),
        pltpu.SemaphoreType.DMA((NBUF,)),
        pltpu.SemaphoreType.DMA((NBUF,)),
        pltpu.SemaphoreType.DMA((NBUF,)),
    ],
)
def _concat_shape(inputs_hbm, idx_hbm, table_hbm, out_hbm,
                  idx_v, buf_v, inp_v, tail_v, sem_g, sem_i, sem_o, sem_t):
    wid = lax.axis_index("s") * NC + lax.axis_index("c")
    lanes = lax.iota(jnp.int32, 16)

    gh = [None] * NBUF
    ih = [None] * NBUF
    oh = [None] * NBUF
    th = [None] * NBUF

    def start_chunk(c):
        b = c % NBUF
        base = wid * B_PER_W + c * CHUNK
        pltpu.sync_copy(idx_hbm.at[pl.ds(base, CHUNK)], idx_v.at[b])
        gh[b] = pltpu.async_copy(
            table_hbm.at[idx_v.at[b], pl.ds(0, ROW_PAD)],
            buf_v.at[b], sem_g.at[b])
        ih[b] = pltpu.async_copy(
            inputs_hbm.at[pl.ds(base, CHUNK)], inp_v.at[b], sem_i.at[b])

    start_chunk(0)
    for c in range(NCHUNK):
        b = c % NBUF
        base = wid * B_PER_W + c * CHUNK
        if c + 1 < NCHUNK:
            nb = (c + 1) % NBUF
            if oh[nb] is not None:
                oh[nb].wait()
                th[nb].wait()
            start_chunk(c + 1)
        gh[b].wait()
        ih[b].wait()

        def body(i, carry, b=b):
            vg = buf_v[b, i, pl.ds(288, 16)]
            rot = _rotate4(inp_v[b, i, pl.ds(0, 16)])
            buf_v[b, i, pl.ds(288, 16)] = jnp.where(lanes < 12, vg, rot)
            for u in range(5):
                buf_v[b, i, pl.ds(304 + 16 * u, 16)] = \
                    inp_v[b, i, pl.ds(4 + 16 * u, 16)]
            tail_v[b, i, pl.ds(0, 16)] = inp_v[b, i, pl.ds(SPLIT, 16)]
            tail_v[b, i, pl.ds(28, 16)] = inp_v[b, i, pl.ds(112, 16)]
            tail_v[b, i, pl.ds(16, 16)] = inp_v[b, i, pl.ds(100, 16)]
            return carry
        lax.fori_loop(0, CHUNK, body, 0)

        oh[b] = pltpu.async_copy(
            buf_v.at[b],
            out_hbm.at[pl.ds(base, CHUNK), pl.ds(0, ROW_PAD)], sem_o.at[b])
        th[b] = pltpu.async_copy(
            tail_v.at[b],
            out_hbm.at[pl.ds(base, CHUNK), pl.ds(ROW_PAD, TAIL)], sem_t.at[b])

    for b in range(NBUF):
        oh[b].wait()
        th[b].wait()


def kernel(inputs, subject_id, s):
    return _concat_shape(inputs, subject_id.astype(jnp.int32), s)
